# Initial kernel scaffold; baseline (speedup 1.0000x reference)
import jax
import jax.numpy as jnp
from jax.experimental import pallas as pl


def kernel(bi_graph, ic_graph, bundles_feature, cates_feature, items_feature):
    raise NotImplementedError("write your pallas kernel here")


# speedup vs baseline: 1.4165x; 1.4165x over previous
import functools

import jax
import jax.numpy as jnp
from jax import lax
from jax.experimental import pallas as pl
from jax.experimental.pallas import tpu as pltpu

NB = 2000
NI = 4000
NC = 500
E = 64

BLK_I = 400
BLK_B = 400
BLK_P = 800

F32 = jnp.float32
BF16 = jnp.bfloat16


def _split_hi_lo(x):
    hi = x.astype(BF16)
    lo = (x - hi.astype(F32)).astype(BF16)
    return hi, lo


def _bdot(a_bf16, x_f32):
    hi, lo = _split_hi_lo(x_f32)
    r = jnp.dot(a_bf16, hi, preferred_element_type=F32)
    r += jnp.dot(a_bf16, lo, preferred_element_type=F32)
    return r


def _l2n(x):
    n = jnp.sqrt(jnp.sum(x * x, axis=1, keepdims=True))
    return x / jnp.maximum(n, 1e-12)


def _inv_sqrt_deg(d):
    return 1.0 / (jnp.sqrt(d) + 1e-8)


def _ii_body(biT_ref, bi_ref, ii_ref, di_ref):
    i = pl.program_id(0)
    cnt = jnp.dot(biT_ref[...], bi_ref[...], preferred_element_type=F32)
    bin_ = (cnt > 0.0)
    ii_ref[...] = bin_.astype(BF16)
    deg = jnp.sum(bin_.astype(F32), axis=1, keepdims=True)
    deg += jnp.sum(biT_ref[...].astype(F32), axis=1, keepdims=True)
    di_ref[pl.ds(i * BLK_I, BLK_I), :] = deg


def _pc_ii(bi_bf, biT_bf):
    grid = NI // BLK_I
    return pl.pallas_call(
        _ii_body,
        grid=(grid,),
        in_specs=[
            pl.BlockSpec((BLK_I, NB), lambda i: (i, 0)),
            pl.BlockSpec((NB, NI), lambda i: (0, 0)),
        ],
        out_specs=[
            pl.BlockSpec((BLK_I, NI), lambda i: (i, 0)),
            pl.BlockSpec((NI, 1), lambda i: (0, 0)),
        ],
        out_shape=[
            jax.ShapeDtypeStruct((NI, NI), BF16),
            jax.ShapeDtypeStruct((NI, 1), F32),
        ],
    )(biT_bf, bi_bf)


def _bb_body(bi_ref, biT_ref, bb_ref, db_ref):
    i = pl.program_id(0)
    cnt = jnp.dot(bi_ref[...], biT_ref[...], preferred_element_type=F32)
    bin_ = (cnt > 0.0)
    bb_ref[...] = bin_.astype(BF16)
    deg = jnp.sum(bin_.astype(F32), axis=1, keepdims=True)
    deg += jnp.sum(bi_ref[...].astype(F32), axis=1, keepdims=True)
    db_ref[pl.ds(i * BLK_B, BLK_B), :] = deg


def _pc_bb(bi_bf, biT_bf):
    grid = NB // BLK_B
    return pl.pallas_call(
        _bb_body,
        grid=(grid,),
        in_specs=[
            pl.BlockSpec((BLK_B, NI), lambda i: (i, 0)),
            pl.BlockSpec((NI, NB), lambda i: (0, 0)),
        ],
        out_specs=[
            pl.BlockSpec((BLK_B, NB), lambda i: (i, 0)),
            pl.BlockSpec((NB, 1), lambda i: (0, 0)),
        ],
        out_shape=[
            jax.ShapeDtypeStruct((NB, NB), BF16),
            jax.ShapeDtypeStruct((NB, 1), F32),
        ],
    )(bi_bf, biT_bf)


def _cate_body(bi_ref, biT_ref, ic_ref, icT_ref, fb_ref, fc_ref, clc_ref):
    bc = jnp.dot(bi_ref[...], ic_ref[...], preferred_element_type=F32)
    bcT = jnp.dot(icT_ref[...], biT_ref[...], preferred_element_type=F32)
    db = jnp.sum(bc, axis=1, keepdims=True)
    dc = jnp.sum(bcT, axis=1, keepdims=True)
    sb = _inv_sqrt_deg(db)
    sc = _inv_sqrt_deg(dc)
    fb = fb_ref[...]
    fc = fc_ref[...]
    u0b = sb * fb
    u0c = sc * fc
    f1b = sb * jnp.dot(bc, u0c, preferred_element_type=F32) * 0.5
    f1c = sc * jnp.dot(bcT, u0b, preferred_element_type=F32) * 0.5
    n1c = _l2n(f1c)
    u1b = sb * f1b
    f2c = sc * jnp.dot(bcT, u1b, preferred_element_type=F32) * (1.0 / 3.0)
    n2c = _l2n(f2c)
    clc_ref[...] = fc + n1c + n2c


def _pc_cate(bi_bf, biT_bf, ic_bf, icT_bf, fb, fc):
    return pl.pallas_call(
        _cate_body,
        out_shape=jax.ShapeDtypeStruct((NC, E), F32),
    )(bi_bf, biT_bf, ic_bf, icT_bf, fb, fc)


def _clitems_body(ic_ref, clc_ref, out_ref):
    out_ref[...] = _bdot(ic_ref[...], clc_ref[...])


def _pc_clitems(ic_bf, clc):
    return pl.pallas_call(
        _clitems_body,
        out_shape=jax.ShapeDtypeStruct((NI, E), F32),
    )(ic_bf, clc)


def _il1b_body(bb_ref, bi_ref, db_ref, dbblk_ref, di_ref, fb_ref, fi_ref,
               u1b_ref, u0b_ref, u0i_ref):
    i = pl.program_id(0)
    sb = _inv_sqrt_deg(db_ref[...])
    si = _inv_sqrt_deg(di_ref[...])
    u0b = sb * fb_ref[...]
    u0i = si * fi_ref[...]

    @pl.when(i == 0)
    def _():
        u0b_ref[...] = u0b
        u0i_ref[...] = u0i

    sb_blk = _inv_sqrt_deg(dbblk_ref[...])
    f1b = sb_blk * (_bdot(bb_ref[...], u0b) + _bdot(bi_ref[...], u0i)) * 0.5
    u1b_ref[...] = sb_blk * f1b


def _pc_il1b(bb_bf, bi_bf, db, di, fb, fi):
    grid = NB // BLK_B
    return pl.pallas_call(
        _il1b_body,
        grid=(grid,),
        in_specs=[
            pl.BlockSpec((BLK_B, NB), lambda i: (i, 0)),
            pl.BlockSpec((BLK_B, NI), lambda i: (i, 0)),
            pl.BlockSpec((NB, 1), lambda i: (0, 0)),
            pl.BlockSpec((BLK_B, 1), lambda i: (i, 0)),
            pl.BlockSpec((NI, 1), lambda i: (0, 0)),
            pl.BlockSpec((NB, E), lambda i: (0, 0)),
            pl.BlockSpec((NI, E), lambda i: (0, 0)),
        ],
        out_specs=[
            pl.BlockSpec((BLK_B, E), lambda i: (i, 0)),
            pl.BlockSpec((NB, E), lambda i: (0, 0)),
            pl.BlockSpec((NI, E), lambda i: (0, 0)),
        ],
        out_shape=[
            jax.ShapeDtypeStruct((NB, E), F32),
            jax.ShapeDtypeStruct((NB, E), F32),
            jax.ShapeDtypeStruct((NI, E), F32),
        ],
    )(bb_bf, bi_bf, db, db, di, fb, fi)


def _il1i_body(biT_ref, ii_ref, u0b_ref, u0i_ref, di_ref, n1i_ref, u1i_ref):
    si = _inv_sqrt_deg(di_ref[...])
    f1i = si * (_bdot(biT_ref[...], u0b_ref[...])
                + _bdot(ii_ref[...], u0i_ref[...])) * 0.5
    n1i_ref[...] = _l2n(f1i)
    u1i_ref[...] = si * f1i


def _pc_il1i(biT_bf, ii_bf, u0b, u0i, di):
    grid = NI // BLK_P
    return pl.pallas_call(
        _il1i_body,
        grid=(grid,),
        in_specs=[
            pl.BlockSpec((BLK_P, NB), lambda i: (i, 0)),
            pl.BlockSpec((BLK_P, NI), lambda i: (i, 0)),
            pl.BlockSpec((NB, E), lambda i: (0, 0)),
            pl.BlockSpec((NI, E), lambda i: (0, 0)),
            pl.BlockSpec((BLK_P, 1), lambda i: (i, 0)),
        ],
        out_specs=[
            pl.BlockSpec((BLK_P, E), lambda i: (i, 0)),
            pl.BlockSpec((BLK_P, E), lambda i: (i, 0)),
        ],
        out_shape=[
            jax.ShapeDtypeStruct((NI, E), F32),
            jax.ShapeDtypeStruct((NI, E), F32),
        ],
    )(biT_bf, ii_bf, u0b, u0i, di)


def _il2_body(biT_ref, ii_ref, u1b_ref, u1i_ref, di_ref, fi_ref, n1i_ref,
              cli_ref, out_ref):
    si = _inv_sqrt_deg(di_ref[...])
    f2i = si * (_bdot(biT_ref[...], u1b_ref[...])
                + _bdot(ii_ref[...], u1i_ref[...])) * (1.0 / 3.0)
    n2i = _l2n(f2i)
    il = fi_ref[...] + n1i_ref[...] + n2i
    out_ref[...] = cli_ref[...] * 0.6 + il * 0.4


def _pc_il2(biT_bf, ii_bf, u1b, u1i, di, fi, n1i, cli):
    grid = NI // BLK_P
    return pl.pallas_call(
        _il2_body,
        grid=(grid,),
        in_specs=[
            pl.BlockSpec((BLK_P, NB), lambda i: (i, 0)),
            pl.BlockSpec((BLK_P, NI), lambda i: (i, 0)),
            pl.BlockSpec((NB, E), lambda i: (0, 0)),
            pl.BlockSpec((NI, E), lambda i: (0, 0)),
            pl.BlockSpec((BLK_P, 1), lambda i: (i, 0)),
            pl.BlockSpec((BLK_P, E), lambda i: (i, 0)),
            pl.BlockSpec((BLK_P, E), lambda i: (i, 0)),
            pl.BlockSpec((BLK_P, E), lambda i: (i, 0)),
        ],
        out_specs=pl.BlockSpec((BLK_P, E), lambda i: (i, 0)),
        out_shape=jax.ShapeDtypeStruct((NI, E), F32),
    )(biT_bf, ii_bf, u1b, u1i, di, fi, n1i, cli)


def kernel(bi_graph, ic_graph, bundles_feature, cates_feature, items_feature):
    bi_bf = bi_graph.astype(BF16)
    biT_bf = bi_bf.T
    ic_bf = ic_graph.astype(BF16)
    icT_bf = ic_bf.T

    ii_bf, di = _pc_ii(bi_bf, biT_bf)
    bb_bf, db = _pc_bb(bi_bf, biT_bf)

    clc = _pc_cate(bi_bf, biT_bf, ic_bf, icT_bf, bundles_feature, cates_feature)
    cli = _pc_clitems(ic_bf, clc)

    u1b, u0b, u0i = _pc_il1b(bb_bf, bi_bf, db, di, bundles_feature, items_feature)
    n1i, u1i = _pc_il1i(biT_bf, ii_bf, u0b, u0i, di)
    out = _pc_il2(biT_bf, ii_bf, u1b, u1i, di, items_feature, n1i, cli)
    return out
